# baseline (speedup 1.0000x reference)
import jax
import jax.numpy as jnp
from jax import lax
from jax.experimental import pallas as pl
from jax.experimental.pallas import tpu as pltpu
from jax.experimental.pallas import tpu_sc as plsc

F_IN = 128
K = 8
F_OUT = 16
KF = K * F_OUT
KP = 16

NC = 2
NS = 16
NW = NC * NS
L = 16

N_TBL = 10240
N_ACC = 10016
EB = 128
BPT = 82
SLOP = 2
CHUNK = BPT * EB
E_PAD = NW * CHUNK
E_EXPE = E_PAD + SLOP * EB
RPT = N_ACC // NS
ZCH = [128, 128, 128, 128, 114]


def _k1_body(x_ref, wt_ref, b_ref, asrc_ref, atrg_ref,
             hp_ref, hs_ref, ht_ref, bound_ref, ms_ref, mt_ref):
    i = pl.program_id(0)
    hp = jnp.dot(x_ref[...], wt_ref[...],
                 preferred_element_type=jnp.float32) + b_ref[...]
    hp_ref[...] = hp
    hs = jnp.dot(hp, asrc_ref[...], preferred_element_type=jnp.float32)
    ht = jnp.dot(hp, atrg_ref[...], preferred_element_type=jnp.float32)
    hs_ref[...] = hs
    ht_ref[...] = ht
    ms = jnp.max(hs)
    mt = jnp.max(ht)

    @pl.when(i == 0)
    def _():
        ms_ref[0] = ms
        mt_ref[0] = mt

    @pl.when(i > 0)
    def _():
        ms_ref[0] = jnp.maximum(ms_ref[0], ms)
        mt_ref[0] = jnp.maximum(mt_ref[0], mt)

    bound_ref[...] = jnp.full((8, 128),
                              jnp.maximum(ms_ref[0] + mt_ref[0], 0.0),
                              dtype=jnp.float32)


def _k1(x_pad, Wt, b2, A_src, A_trg):
    grid = N_TBL // 1024
    return pl.pallas_call(
        _k1_body,
        grid=(grid,),
        in_specs=[
            pl.BlockSpec((1024, F_IN), lambda i: (i, 0)),
            pl.BlockSpec((F_IN, KF), lambda i: (0, 0)),
            pl.BlockSpec((1, KF), lambda i: (0, 0)),
            pl.BlockSpec((KF, KP), lambda i: (0, 0)),
            pl.BlockSpec((KF, KP), lambda i: (0, 0)),
        ],
        out_specs=[
            pl.BlockSpec((1024, KF), lambda i: (i, 0)),
            pl.BlockSpec((1024, KP), lambda i: (i, 0)),
            pl.BlockSpec((1024, KP), lambda i: (i, 0)),
            pl.BlockSpec((8, 128), lambda i: (0, 0)),
        ],
        out_shape=[
            jax.ShapeDtypeStruct((N_TBL, KF), jnp.float32),
            jax.ShapeDtypeStruct((N_TBL, KP), jnp.float32),
            jax.ShapeDtypeStruct((N_TBL, KP), jnp.float32),
            jax.ShapeDtypeStruct((8, 128), jnp.float32),
        ],
        scratch_shapes=[pltpu.SMEM((1,), jnp.float32),
                        pltpu.SMEM((1,), jnp.float32)],
    )(x_pad, Wt, b2, A_src, A_trg)


def _k2_body(src2, trg2, hs_hbm, ht_hbm, bound_hbm,
             expe_hbm, norm_hbm,
             src_v, trg_v, s0, s1, t0, t1, e0, e1, bound_v,
             norm_acc, sem0, sem1):
    cid = lax.axis_index("c")
    sid = lax.axis_index("s")
    wid = cid * NS + sid
    sbuf, tbuf, ebuf, sems = [s0, s1], [t0, t1], [e0, e1], [sem0, sem1]

    @pl.loop(0, EB)
    def _(r):
        e0[r, :] = jnp.zeros((KP,), jnp.float32)

    zoff = 0
    for zc in ZCH:
        pltpu.sync_copy(e0.at[pl.ds(0, zc), :],
                        norm_acc.at[pl.ds(sid * RPT + zoff, zc), :])
        zoff += zc

    pltpu.sync_copy(bound_hbm.at[0, pl.ds(0, L)], bound_v)
    pltpu.sync_copy(src2.at[wid], src_v)
    pltpu.sync_copy(trg2.at[wid], trg_v)
    plsc.subcore_barrier()

    def issue(blk, b):
        pltpu.async_copy(hs_hbm.at[src_v.at[blk]], sbuf[b], sems[b])
        pltpu.async_copy(ht_hbm.at[trg_v.at[blk]], tbuf[b], sems[b])

    def wait_pair(b):
        pltpu.make_async_copy(hs_hbm.at[pl.ds(0, EB), :], sbuf[b],
                              sems[b]).wait()
        pltpu.make_async_copy(ht_hbm.at[pl.ds(0, EB), :], tbuf[b],
                              sems[b]).wait()

    issue(0, 0)
    issue(1, 1)
    bound = bound_v[...]

    @pl.loop(0, BPT, step=2)
    def _(blk0):
        for b in range(2):
            blk = blk0 + b
            wait_pair(b)
            sb, tb, eb = sbuf[b], tbuf[b], ebuf[b]

            @pl.loop(0, EB)
            def _(ei):
                e = sb[ei, :] + tb[ei, :]
                e = jnp.where(e > 0, e, 0.2 * e) - bound
                eb[ei, :] = jnp.exp(e)

            off = wid * CHUNK + blk * EB
            pltpu.sync_copy(eb, expe_hbm.at[pl.ds(off, EB), :])
            pltpu.sync_copy(eb, norm_acc.at[trg_v.at[blk]], add=True)
            issue(blk + 2, b)

    for b in range(2):
        wait_pair(b)
    plsc.subcore_barrier()
    pltpu.sync_copy(norm_acc.at[pl.ds(sid * RPT, RPT), :],
                    norm_hbm.at[cid, pl.ds(sid * RPT, RPT), :])


def _k2(src2, trg2, hs, ht, bound):
    mesh = plsc.VectorSubcoreMesh(core_axis_name="c", subcore_axis_name="s")
    f = pl.kernel(
        _k2_body,
        out_type=[
            jax.ShapeDtypeStruct((E_EXPE, KP), jnp.float32),
            jax.ShapeDtypeStruct((NC, N_ACC, KP), jnp.float32),
        ],
        mesh=mesh,
        compiler_params=pltpu.CompilerParams(use_tc_tiling_on_sc=False),
        scratch_types=[
            pltpu.VMEM((BPT + SLOP, EB), jnp.int32),
            pltpu.VMEM((BPT + SLOP, EB), jnp.int32),
            pltpu.VMEM((EB, KP), jnp.float32),
            pltpu.VMEM((EB, KP), jnp.float32),
            pltpu.VMEM((EB, KP), jnp.float32),
            pltpu.VMEM((EB, KP), jnp.float32),
            pltpu.VMEM((EB, KP), jnp.float32),
            pltpu.VMEM((EB, KP), jnp.float32),
            pltpu.VMEM((L,), jnp.float32),
            pltpu.VMEM_SHARED((N_ACC, KP), jnp.float32),
            pltpu.SemaphoreType.DMA,
            pltpu.SemaphoreType.DMA,
        ],
    )
    return f(src2, trg2, hs, ht, bound)


PH_A = 40
PW = BPT - PH_A + SLOP


def _k4_body(src2, trg2, expe_hbm, hp_hbm,
             out_hbm,
             spg, tpg, e0, e1, h0, h1,
             out_acc, sem0, sem1):
    cid = lax.axis_index("c")
    sid = lax.axis_index("s")
    wid = cid * NS + sid
    ebuf, hbuf, sems = [e0, e1], [h0, h1], [sem0, sem1]

    @pl.loop(0, EB)
    def _(r):
        for j in range(KF // L):
            h0[r, pl.ds(j * L, L)] = jnp.zeros((L,), jnp.float32)

    zoff = 0
    for zc in ZCH:
        pltpu.sync_copy(h0.at[pl.ds(0, zc), :],
                        out_acc.at[pl.ds(sid * RPT + zoff, zc), :])
        zoff += zc

    pltpu.sync_copy(src2.at[wid, pl.ds(0, PW), :], spg)
    pltpu.sync_copy(trg2.at[wid, pl.ds(0, PW), :], tpg)
    plsc.subcore_barrier()

    def issue(blk, row, b):
        off = wid * CHUNK + blk * EB
        pltpu.async_copy(expe_hbm.at[pl.ds(off, EB), :], ebuf[b], sems[b])
        pltpu.async_copy(hp_hbm.at[spg.at[row]], hbuf[b], sems[b])

    def wait_pair(b):
        pltpu.make_async_copy(expe_hbm.at[pl.ds(0, EB), :], ebuf[b],
                              sems[b]).wait()
        pltpu.make_async_copy(hp_hbm.at[pl.ds(0, EB), :], hbuf[b],
                              sems[b]).wait()

    splat_idx = [jnp.full((L,), k, jnp.int32) for k in range(K)]

    def step(blk, row, b):
        wait_pair(b)
        eb, hb = ebuf[b], hbuf[b]

        @pl.loop(0, EB)
        def _(ei):
            av = eb[ei, :]
            for k in range(K):
                seg = pl.ds(k * F_OUT, F_OUT)
                aspl = av.at[splat_idx[k]].get(mode="promise_in_bounds")
                hb[ei, seg] = hb[ei, seg] * aspl

        pltpu.sync_copy(hb, out_acc.at[tpg.at[row]], add=True)

    issue(0, 0, 0)
    issue(1, 1, 1)

    @pl.loop(0, PH_A, step=2)
    def _(blk0):
        for b in range(2):
            blk = blk0 + b
            step(blk, blk, b)
            issue(blk + 2, blk + 2, b)

    for b in range(2):
        wait_pair(b)
    pltpu.sync_copy(src2.at[wid, pl.ds(PH_A, PW), :], spg)
    pltpu.sync_copy(trg2.at[wid, pl.ds(PH_A, PW), :], tpg)

    @pl.loop(PH_A, BPT, step=2)
    def _(blk0):
        for b in range(2):
            blk = blk0 + b
            row = blk - PH_A

            @pl.when(blk0 > PH_A)
            def _():
                wait_pair(b)

            eb, hb = ebuf[b], hbuf[b]

            @pl.loop(0, EB)
            def _(ei):
                av = eb[ei, :]
                for k in range(K):
                    seg = pl.ds(k * F_OUT, F_OUT)
                    aspl = av.at[splat_idx[k]].get(
                        mode="promise_in_bounds")
                    hb[ei, seg] = hb[ei, seg] * aspl

            pltpu.sync_copy(hb, out_acc.at[tpg.at[row]], add=True)
            issue(blk + 2, row + 2, b)

    for b in range(2):
        wait_pair(b)
    plsc.subcore_barrier()
    pltpu.sync_copy(out_acc.at[pl.ds(sid * RPT, RPT), :],
                    out_hbm.at[cid, pl.ds(sid * RPT, RPT), :])


def _k4(src2, trg2, expe, hp):
    mesh = plsc.VectorSubcoreMesh(core_axis_name="c", subcore_axis_name="s")
    f = pl.kernel(
        _k4_body,
        out_type=[
            jax.ShapeDtypeStruct((NC, N_ACC, KF), jnp.float32),
        ],
        mesh=mesh,
        compiler_params=pltpu.CompilerParams(use_tc_tiling_on_sc=False),
        scratch_types=[
            pltpu.VMEM((PW, EB), jnp.int32),
            pltpu.VMEM((PW, EB), jnp.int32),
            pltpu.VMEM((EB, KP), jnp.float32),
            pltpu.VMEM((EB, KP), jnp.float32),
            pltpu.VMEM((EB, KF), jnp.float32),
            pltpu.VMEM((EB, KF), jnp.float32),
            pltpu.VMEM_SHARED((N_ACC, KF), jnp.float32),
            pltpu.SemaphoreType.DMA,
            pltpu.SemaphoreType.DMA,
        ],
    )
    return f(src2, trg2, expe, hp)


def _k5_body(p_ref, n_ref, exp_ref, out_ref):
    rn = 1.0 / (n_ref[0, :, :K] + n_ref[1, :, :K] + 1e-10)
    scale = jnp.dot(rn, exp_ref[...], preferred_element_type=jnp.float32)
    out_ref[...] = (p_ref[0] + p_ref[1]) * scale


def _k5(out_part, norm_part, one_hot):
    return pl.pallas_call(
        _k5_body,
        out_shape=jax.ShapeDtypeStruct((N_ACC, KF), jnp.float32),
    )(out_part, norm_part, one_hot)


def kernel(x, edge_index, W, b, a_src, a_trg):
    n = x.shape[0]
    e = edge_index.shape[1]
    e_tot = e + n
    self_loop = jnp.arange(n, dtype=edge_index.dtype)
    pad_idx = jnp.full((E_PAD - e_tot,), n, dtype=edge_index.dtype)
    src = jnp.concatenate([edge_index[0], self_loop, pad_idx])
    trg = jnp.concatenate([edge_index[1], self_loop, pad_idx])
    src3 = src.reshape(NW, BPT, EB)
    trg3 = trg.reshape(NW, BPT, EB)
    src2 = jnp.pad(src3, ((0, 0), (0, SLOP), (0, 0)))
    trg2 = jnp.pad(trg3, ((0, 0), (0, SLOP), (0, 0)))

    x_pad = jnp.pad(x, ((0, N_TBL - n), (0, 0)))
    Wt = W.T
    b2 = b.reshape(1, KF)
    rows = jnp.arange(KF)
    cols = rows // F_OUT
    A_src = jnp.zeros((KF, KP), jnp.float32).at[rows, cols].set(
        a_src.reshape(KF))
    A_trg = jnp.zeros((KF, KP), jnp.float32).at[rows, cols].set(
        a_trg.reshape(KF))
    one_hot = jnp.zeros((K, KF), jnp.float32).at[cols, rows].set(1.0)

    hp, hs, ht, bound = _k1(x_pad, Wt, b2, A_src, A_trg)
    expe, norm_part = _k2(src2, trg2, hs, ht, bound)
    (out_part,) = _k4(src2, trg2, expe, hp)
    out = _k5(out_part, norm_part, one_hot)
    return out[:n]


# speedup vs baseline: 1.0728x; 1.0728x over previous
import jax
import jax.numpy as jnp
from jax import lax
from jax.experimental import pallas as pl
from jax.experimental.pallas import tpu as pltpu
from jax.experimental.pallas import tpu_sc as plsc

F_IN = 128
K = 8
F_OUT = 16
KF = K * F_OUT
KP = 16

NC = 2
NS = 16
NW = NC * NS
L = 16

N_PAD = 10240
EB = 128
BPT = 82
EB4 = 64
BPT4 = 164
C0 = 180
C1 = 2 * BPT4 - C0
SLOP = 2
CHUNK = BPT * EB
E_PAD = NW * CHUNK
E_EXPE = E_PAD + SLOP * EB4
RPT = N_PAD // NS


def _k1_body(x_ref, wt_ref, b_ref, asrc_ref, atrg_ref,
             hp_ref, hs_ref, ht_ref, bound_ref, ms_ref, mt_ref):
    i = pl.program_id(0)
    hp = jnp.dot(x_ref[...], wt_ref[...],
                 preferred_element_type=jnp.float32) + b_ref[...]
    hp_ref[...] = hp
    hs = jnp.dot(hp, asrc_ref[...], preferred_element_type=jnp.float32)
    ht = jnp.dot(hp, atrg_ref[...], preferred_element_type=jnp.float32)
    hs_ref[...] = hs
    ht_ref[...] = ht
    ms = jnp.max(hs)
    mt = jnp.max(ht)

    @pl.when(i == 0)
    def _():
        ms_ref[0] = ms
        mt_ref[0] = mt

    @pl.when(i > 0)
    def _():
        ms_ref[0] = jnp.maximum(ms_ref[0], ms)
        mt_ref[0] = jnp.maximum(mt_ref[0], mt)

    bound_ref[...] = jnp.full((8, 128),
                              jnp.maximum(ms_ref[0] + mt_ref[0], 0.0),
                              dtype=jnp.float32)


def _k1(x_pad, Wt, b2, A_src, A_trg):
    grid = N_PAD // 1024
    return pl.pallas_call(
        _k1_body,
        grid=(grid,),
        in_specs=[
            pl.BlockSpec((1024, F_IN), lambda i: (i, 0)),
            pl.BlockSpec((F_IN, KF), lambda i: (0, 0)),
            pl.BlockSpec((1, KF), lambda i: (0, 0)),
            pl.BlockSpec((KF, KP), lambda i: (0, 0)),
            pl.BlockSpec((KF, KP), lambda i: (0, 0)),
        ],
        out_specs=[
            pl.BlockSpec((1024, KF), lambda i: (i, 0)),
            pl.BlockSpec((1024, KP), lambda i: (i, 0)),
            pl.BlockSpec((1024, KP), lambda i: (i, 0)),
            pl.BlockSpec((8, 128), lambda i: (0, 0)),
        ],
        out_shape=[
            jax.ShapeDtypeStruct((N_PAD, KF), jnp.float32),
            jax.ShapeDtypeStruct((N_PAD, KP), jnp.float32),
            jax.ShapeDtypeStruct((N_PAD, KP), jnp.float32),
            jax.ShapeDtypeStruct((8, 128), jnp.float32),
        ],
        scratch_shapes=[pltpu.SMEM((1,), jnp.float32),
                        pltpu.SMEM((1,), jnp.float32)],
    )(x_pad, Wt, b2, A_src, A_trg)


def _k2_body(src2, trg2, hs_hbm, ht_hbm, bound_hbm,
             expe_hbm, norm_hbm,
             src_v, trg_v, s0, s1, t0, t1, e0, e1, bound_v,
             norm_acc, sem0, sem1):
    cid = lax.axis_index("c")
    sid = lax.axis_index("s")
    wid = cid * NS + sid
    sbuf, tbuf, ebuf, sems = [s0, s1], [t0, t1], [e0, e1], [sem0, sem1]

    @pl.loop(0, EB)
    def _(r):
        e0[r, :] = jnp.zeros((KP,), jnp.float32)

    @pl.loop(0, RPT // EB)
    def _(r):
        pltpu.sync_copy(e0, norm_acc.at[pl.ds(sid * RPT + r * EB, EB), :])

    pltpu.sync_copy(bound_hbm.at[0, pl.ds(0, L)], bound_v)
    pltpu.sync_copy(src2.at[wid], src_v)
    pltpu.sync_copy(trg2.at[wid], trg_v)
    plsc.subcore_barrier()

    def issue(blk, b):
        pltpu.async_copy(hs_hbm.at[src_v.at[blk]], sbuf[b], sems[b])
        pltpu.async_copy(ht_hbm.at[trg_v.at[blk]], tbuf[b], sems[b])

    def wait_pair(b):
        pltpu.make_async_copy(hs_hbm.at[pl.ds(0, EB), :], sbuf[b],
                              sems[b]).wait()
        pltpu.make_async_copy(ht_hbm.at[pl.ds(0, EB), :], tbuf[b],
                              sems[b]).wait()

    issue(0, 0)
    issue(1, 1)
    bound = bound_v[...]

    @pl.loop(0, BPT, step=2)
    def _(blk0):
        for b in range(2):
            blk = blk0 + b
            wait_pair(b)
            sb, tb, eb = sbuf[b], tbuf[b], ebuf[b]

            @pl.loop(0, EB)
            def _(ei):
                e = sb[ei, :] + tb[ei, :]
                e = jnp.where(e > 0, e, 0.2 * e) - bound
                eb[ei, :] = jnp.exp(e)

            off = wid * CHUNK + blk * EB
            pltpu.sync_copy(eb, expe_hbm.at[pl.ds(off, EB), :])
            pltpu.sync_copy(eb, norm_acc.at[trg_v.at[blk]], add=True)
            issue(blk + 2, b)

    for b in range(2):
        wait_pair(b)
    plsc.subcore_barrier()
    pltpu.sync_copy(norm_acc.at[pl.ds(sid * RPT, RPT), :],
                    norm_hbm.at[cid, pl.ds(sid * RPT, RPT), :])


def _k2(src2, trg2, hs, ht, bound):
    mesh = plsc.VectorSubcoreMesh(core_axis_name="c", subcore_axis_name="s")
    f = pl.kernel(
        _k2_body,
        out_type=[
            jax.ShapeDtypeStruct((E_EXPE, KP), jnp.float32),
            jax.ShapeDtypeStruct((NC, N_PAD, KP), jnp.float32),
        ],
        mesh=mesh,
        compiler_params=pltpu.CompilerParams(use_tc_tiling_on_sc=False),
        scratch_types=[
            pltpu.VMEM((BPT + SLOP, EB), jnp.int32),
            pltpu.VMEM((BPT + SLOP, EB), jnp.int32),
            pltpu.VMEM((EB, KP), jnp.float32),
            pltpu.VMEM((EB, KP), jnp.float32),
            pltpu.VMEM((EB, KP), jnp.float32),
            pltpu.VMEM((EB, KP), jnp.float32),
            pltpu.VMEM((EB, KP), jnp.float32),
            pltpu.VMEM((EB, KP), jnp.float32),
            pltpu.VMEM((L,), jnp.float32),
            pltpu.VMEM_SHARED((N_PAD, KP), jnp.float32),
            pltpu.SemaphoreType.DMA,
            pltpu.SemaphoreType.DMA,
        ],
    )
    return f(src2, trg2, hs, ht, bound)


def _k3_body(n_ref, out_ref):
    out_ref[...] = 1.0 / (n_ref[0] + n_ref[1] + 1e-10)


def _k3(norm_part):
    return pl.pallas_call(
        _k3_body,
        out_shape=jax.ShapeDtypeStruct((N_PAD, KP), jnp.float32),
    )(norm_part)


def _k4_body(src2, trg2, expe_hbm, rnorm_hbm, hp_hbm,
             out_hbm,
             src_v, trg_v, rn0, rn1, e0, e1, h0, h1,
             out_acc, sem0, sem1):
    cid = lax.axis_index("c")
    sid = lax.axis_index("s")
    wid = cid * NS + sid
    nblk = jnp.where(cid == 0, C0, C1)
    eblk = jnp.where(cid == 0, sid * C0, NS * C0 + sid * C1)
    rnbuf, ebuf, sems = [rn0, rn1], [e0, e1], [sem0, sem1]
    hbuf = [h0, h1]

    @pl.loop(0, EB4)
    def _(r):
        for j in range(KF // L):
            h0[r, pl.ds(j * L, L)] = jnp.zeros((L,), jnp.float32)

    @pl.loop(0, RPT // EB4)
    def _(r):
        pltpu.sync_copy(h0, out_acc.at[pl.ds(sid * RPT + r * EB4, EB4), :])

    pltpu.sync_copy(src2.at[wid], src_v)
    pltpu.sync_copy(trg2.at[wid], trg_v)
    plsc.subcore_barrier()

    def issue(blk, b):
        off = (eblk + blk) * EB4
        pltpu.async_copy(rnorm_hbm.at[trg_v.at[blk]], rnbuf[b], sems[b])
        pltpu.async_copy(expe_hbm.at[pl.ds(off, EB4), :], ebuf[b], sems[b])
        pltpu.async_copy(hp_hbm.at[src_v.at[blk]], hbuf[b], sems[b])

    def wait_three(b):
        pltpu.make_async_copy(rnorm_hbm.at[pl.ds(0, EB4), :], rnbuf[b],
                              sems[b]).wait()
        pltpu.make_async_copy(expe_hbm.at[pl.ds(0, EB4), :], ebuf[b],
                              sems[b]).wait()
        pltpu.make_async_copy(hp_hbm.at[pl.ds(0, EB4), :], hbuf[b],
                              sems[b]).wait()

    issue(0, 0)
    issue(1, 1)
    splat_idx = [jnp.full((L,), k, jnp.int32) for k in range(K)]

    @pl.loop(0, nblk, step=2)
    def _(blk0):
        for b in range(2):
            blk = blk0 + b
            wait_three(b)
            rnb, eb, hb = rnbuf[b], ebuf[b], hbuf[b]

            @pl.loop(0, EB4)
            def _(ei):
                av = rnb[ei, :] * eb[ei, :]
                for k in range(K):
                    seg = pl.ds(k * F_OUT, F_OUT)
                    aspl = av.at[splat_idx[k]].get(
                        mode="promise_in_bounds")
                    hb[ei, seg] = hb[ei, seg] * aspl

            pltpu.sync_copy(hb, out_acc.at[trg_v.at[blk]], add=True)
            issue(blk + 2, b)

    for b in range(2):
        wait_three(b)
    plsc.subcore_barrier()
    pltpu.sync_copy(out_acc.at[pl.ds(sid * RPT, RPT), :],
                    out_hbm.at[cid, pl.ds(sid * RPT, RPT), :])


def _k4(src2, trg2, expe, rnorm, hp):
    mesh = plsc.VectorSubcoreMesh(core_axis_name="c", subcore_axis_name="s")
    f = pl.kernel(
        _k4_body,
        out_type=[
            jax.ShapeDtypeStruct((NC, N_PAD, KF), jnp.float32),
        ],
        mesh=mesh,
        compiler_params=pltpu.CompilerParams(use_tc_tiling_on_sc=False),
        scratch_types=[
            pltpu.VMEM((C0 + SLOP, EB4), jnp.int32),
            pltpu.VMEM((C0 + SLOP, EB4), jnp.int32),
            pltpu.VMEM((EB4, KP), jnp.float32),
            pltpu.VMEM((EB4, KP), jnp.float32),
            pltpu.VMEM((EB4, KP), jnp.float32),
            pltpu.VMEM((EB4, KP), jnp.float32),
            pltpu.VMEM((EB4, KF), jnp.float32),
            pltpu.VMEM((EB4, KF), jnp.float32),
            pltpu.VMEM_SHARED((N_PAD, KF), jnp.float32),
            pltpu.SemaphoreType.DMA,
            pltpu.SemaphoreType.DMA,
        ],
    )
    return f(src2, trg2, expe, rnorm, hp)


def _k5_body(p_ref, out_ref):
    out_ref[...] = p_ref[0] + p_ref[1]


def _k5(out_part):
    grid = N_PAD // 1024
    return pl.pallas_call(
        _k5_body,
        grid=(grid,),
        in_specs=[pl.BlockSpec((2, 1024, KF), lambda i: (0, i, 0))],
        out_specs=pl.BlockSpec((1024, KF), lambda i: (i, 0)),
        out_shape=jax.ShapeDtypeStruct((N_PAD, KF), jnp.float32),
    )(out_part)


def kernel(x, edge_index, W, b, a_src, a_trg):
    n = x.shape[0]
    e = edge_index.shape[1]
    e_tot = e + n
    self_loop = jnp.arange(n, dtype=edge_index.dtype)
    pad_idx = jnp.full((E_PAD - e_tot,), n, dtype=edge_index.dtype)
    src = jnp.concatenate([edge_index[0], self_loop, pad_idx])
    trg = jnp.concatenate([edge_index[1], self_loop, pad_idx])
    src2 = jnp.pad(src.reshape(NW, BPT, EB), ((0, 0), (0, SLOP), (0, 0)))
    trg2 = jnp.pad(trg.reshape(NW, BPT, EB), ((0, 0), (0, SLOP), (0, 0)))
    def _asym_pages(v):
        cut = NS * C0 * EB4
        p0 = v[:cut].reshape(NS, C0, EB4)
        p1 = v[cut:].reshape(NS, C1, EB4)
        p0 = jnp.pad(p0, ((0, 0), (0, SLOP), (0, 0)))
        p1 = jnp.pad(p1, ((0, 0), (0, C0 - C1 + SLOP), (0, 0)))
        return jnp.concatenate([p0, p1], axis=0)

    src4 = _asym_pages(src)
    trg4 = _asym_pages(trg)

    x_pad = jnp.pad(x, ((0, N_PAD - n), (0, 0)))
    Wt = W.T
    b2 = b.reshape(1, KF)
    rows = jnp.arange(KF)
    cols = rows // F_OUT
    A_src = jnp.zeros((KF, KP), jnp.float32).at[rows, cols].set(
        a_src.reshape(KF))
    A_trg = jnp.zeros((KF, KP), jnp.float32).at[rows, cols].set(
        a_trg.reshape(KF))

    hp, hs, ht, bound = _k1(x_pad, Wt, b2, A_src, A_trg)
    expe, norm_part = _k2(src2, trg2, hs, ht, bound)
    rnorm = _k3(norm_part)
    (out_part,) = _k4(src4, trg4, expe, rnorm, hp)
    out = _k5(out_part)
    return out[:n]


# speedup vs baseline: 1.0781x; 1.0049x over previous
import jax
import jax.numpy as jnp
from jax import lax
from jax.experimental import pallas as pl
from jax.experimental.pallas import tpu as pltpu
from jax.experimental.pallas import tpu_sc as plsc

F_IN = 128
K = 8
F_OUT = 16
KF = K * F_OUT
KP = 16

NC = 2
NS = 16
NW = NC * NS
L = 16

N_PAD = 10240
EB = 128
BPT = 82
EB4 = 64
BPT4 = 164
C0 = 182
C1 = 2 * BPT4 - C0
SLOP = 2
CHUNK = BPT * EB
E_PAD = NW * CHUNK
E_EXPE = E_PAD + SLOP * EB4
RPT = N_PAD // NS


def _k1_body(x_ref, wt_ref, b_ref, asrc_ref, atrg_ref,
             hp_ref, hs_ref, ht_ref, bound_ref, ms_ref, mt_ref):
    i = pl.program_id(0)
    hp = jnp.dot(x_ref[...], wt_ref[...],
                 preferred_element_type=jnp.float32) + b_ref[...]
    hp_ref[...] = hp
    hs = jnp.dot(hp, asrc_ref[...], preferred_element_type=jnp.float32)
    ht = jnp.dot(hp, atrg_ref[...], preferred_element_type=jnp.float32)
    hs_ref[...] = hs
    ht_ref[...] = ht
    ms = jnp.max(hs)
    mt = jnp.max(ht)

    @pl.when(i == 0)
    def _():
        ms_ref[0] = ms
        mt_ref[0] = mt

    @pl.when(i > 0)
    def _():
        ms_ref[0] = jnp.maximum(ms_ref[0], ms)
        mt_ref[0] = jnp.maximum(mt_ref[0], mt)

    bound_ref[...] = jnp.full((8, 128),
                              jnp.maximum(ms_ref[0] + mt_ref[0], 0.0),
                              dtype=jnp.float32)


def _k1(x_pad, Wt, b2, A_src, A_trg):
    grid = N_PAD // 1024
    return pl.pallas_call(
        _k1_body,
        grid=(grid,),
        in_specs=[
            pl.BlockSpec((1024, F_IN), lambda i: (i, 0)),
            pl.BlockSpec((F_IN, KF), lambda i: (0, 0)),
            pl.BlockSpec((1, KF), lambda i: (0, 0)),
            pl.BlockSpec((KF, KP), lambda i: (0, 0)),
            pl.BlockSpec((KF, KP), lambda i: (0, 0)),
        ],
        out_specs=[
            pl.BlockSpec((1024, KF), lambda i: (i, 0)),
            pl.BlockSpec((1024, KP), lambda i: (i, 0)),
            pl.BlockSpec((1024, KP), lambda i: (i, 0)),
            pl.BlockSpec((8, 128), lambda i: (0, 0)),
        ],
        out_shape=[
            jax.ShapeDtypeStruct((N_PAD, KF), jnp.float32),
            jax.ShapeDtypeStruct((N_PAD, KP), jnp.float32),
            jax.ShapeDtypeStruct((N_PAD, KP), jnp.float32),
            jax.ShapeDtypeStruct((8, 128), jnp.float32),
        ],
        scratch_shapes=[pltpu.SMEM((1,), jnp.float32),
                        pltpu.SMEM((1,), jnp.float32)],
    )(x_pad, Wt, b2, A_src, A_trg)


def _k2_body(src2, trg2, hs_hbm, ht_hbm, bound_hbm,
             expe_hbm, norm_hbm,
             src_v, trg_v, s0, s1, t0, t1, e0, e1, bound_v,
             norm_acc, sem0, sem1):
    cid = lax.axis_index("c")
    sid = lax.axis_index("s")
    wid = cid * NS + sid
    sbuf, tbuf, ebuf, sems = [s0, s1], [t0, t1], [e0, e1], [sem0, sem1]

    @pl.loop(0, EB)
    def _(r):
        e0[r, :] = jnp.zeros((KP,), jnp.float32)

    @pl.loop(0, RPT // EB)
    def _(r):
        pltpu.sync_copy(e0, norm_acc.at[pl.ds(sid * RPT + r * EB, EB), :])

    pltpu.sync_copy(bound_hbm.at[0, pl.ds(0, L)], bound_v)
    pltpu.sync_copy(src2.at[wid], src_v)
    pltpu.sync_copy(trg2.at[wid], trg_v)
    plsc.subcore_barrier()

    def issue(blk, b):
        pltpu.async_copy(hs_hbm.at[src_v.at[blk]], sbuf[b], sems[b])
        pltpu.async_copy(ht_hbm.at[trg_v.at[blk]], tbuf[b], sems[b])

    def wait_pair(b):
        pltpu.make_async_copy(hs_hbm.at[pl.ds(0, EB), :], sbuf[b],
                              sems[b]).wait()
        pltpu.make_async_copy(ht_hbm.at[pl.ds(0, EB), :], tbuf[b],
                              sems[b]).wait()

    issue(0, 0)
    issue(1, 1)
    bound = bound_v[...]

    @pl.loop(0, BPT, step=2)
    def _(blk0):
        for b in range(2):
            blk = blk0 + b
            wait_pair(b)
            sb, tb, eb = sbuf[b], tbuf[b], ebuf[b]

            @pl.loop(0, EB)
            def _(ei):
                e = sb[ei, :] + tb[ei, :]
                e = jnp.where(e > 0, e, 0.2 * e) - bound
                eb[ei, :] = jnp.exp(e)

            off = wid * CHUNK + blk * EB
            pltpu.sync_copy(eb, expe_hbm.at[pl.ds(off, EB), :])
            pltpu.sync_copy(eb, norm_acc.at[trg_v.at[blk]], add=True)
            issue(blk + 2, b)

    for b in range(2):
        wait_pair(b)
    plsc.subcore_barrier()
    pltpu.sync_copy(norm_acc.at[pl.ds(sid * RPT, RPT), :],
                    norm_hbm.at[cid, pl.ds(sid * RPT, RPT), :])


def _k2(src2, trg2, hs, ht, bound):
    mesh = plsc.VectorSubcoreMesh(core_axis_name="c", subcore_axis_name="s")
    f = pl.kernel(
        _k2_body,
        out_type=[
            jax.ShapeDtypeStruct((E_EXPE, KP), jnp.float32),
            jax.ShapeDtypeStruct((NC, N_PAD, KP), jnp.float32),
        ],
        mesh=mesh,
        compiler_params=pltpu.CompilerParams(use_tc_tiling_on_sc=False),
        scratch_types=[
            pltpu.VMEM((BPT + SLOP, EB), jnp.int32),
            pltpu.VMEM((BPT + SLOP, EB), jnp.int32),
            pltpu.VMEM((EB, KP), jnp.float32),
            pltpu.VMEM((EB, KP), jnp.float32),
            pltpu.VMEM((EB, KP), jnp.float32),
            pltpu.VMEM((EB, KP), jnp.float32),
            pltpu.VMEM((EB, KP), jnp.float32),
            pltpu.VMEM((EB, KP), jnp.float32),
            pltpu.VMEM((L,), jnp.float32),
            pltpu.VMEM_SHARED((N_PAD, KP), jnp.float32),
            pltpu.SemaphoreType.DMA,
            pltpu.SemaphoreType.DMA,
        ],
    )
    return f(src2, trg2, hs, ht, bound)


def _k3_body(n_ref, out_ref):
    out_ref[...] = 1.0 / (n_ref[0] + n_ref[1] + 1e-10)


def _k3(norm_part):
    return pl.pallas_call(
        _k3_body,
        out_shape=jax.ShapeDtypeStruct((N_PAD, KP), jnp.float32),
    )(norm_part)


def _k4_body(src2, trg2, expe_hbm, rnorm_hbm, hp_hbm,
             out_hbm,
             src_v, trg_v, rn0, rn1, e0, e1, h0, h1,
             out_acc, sem0, sem1):
    cid = lax.axis_index("c")
    sid = lax.axis_index("s")
    wid = cid * NS + sid
    nblk = jnp.where(cid == 0, C0, C1)
    eblk = jnp.where(cid == 0, sid * C0, NS * C0 + sid * C1)
    rnbuf, ebuf, sems = [rn0, rn1], [e0, e1], [sem0, sem1]
    hbuf = [h0, h1]

    @pl.loop(0, EB4)
    def _(r):
        for j in range(KF // L):
            h0[r, pl.ds(j * L, L)] = jnp.zeros((L,), jnp.float32)

    @pl.loop(0, RPT // EB4)
    def _(r):
        pltpu.sync_copy(h0, out_acc.at[pl.ds(sid * RPT + r * EB4, EB4), :])

    pltpu.sync_copy(src2.at[wid], src_v)
    pltpu.sync_copy(trg2.at[wid], trg_v)
    plsc.subcore_barrier()

    def issue(blk, b):
        off = (eblk + blk) * EB4
        pltpu.async_copy(rnorm_hbm.at[trg_v.at[blk]], rnbuf[b], sems[b])
        pltpu.async_copy(expe_hbm.at[pl.ds(off, EB4), :], ebuf[b], sems[b])
        pltpu.async_copy(hp_hbm.at[src_v.at[blk]], hbuf[b], sems[b])

    def wait_three(b):
        pltpu.make_async_copy(rnorm_hbm.at[pl.ds(0, EB4), :], rnbuf[b],
                              sems[b]).wait()
        pltpu.make_async_copy(expe_hbm.at[pl.ds(0, EB4), :], ebuf[b],
                              sems[b]).wait()
        pltpu.make_async_copy(hp_hbm.at[pl.ds(0, EB4), :], hbuf[b],
                              sems[b]).wait()

    issue(0, 0)
    issue(1, 1)
    splat_idx = [jnp.full((L,), k, jnp.int32) for k in range(K)]

    @pl.loop(0, nblk, step=2)
    def _(blk0):
        for b in range(2):
            blk = blk0 + b
            wait_three(b)
            rnb, eb, hb = rnbuf[b], ebuf[b], hbuf[b]

            @pl.loop(0, EB4)
            def _(ei):
                av = rnb[ei, :] * eb[ei, :]
                for k in range(K):
                    seg = pl.ds(k * F_OUT, F_OUT)
                    aspl = av.at[splat_idx[k]].get(
                        mode="promise_in_bounds")
                    hb[ei, seg] = hb[ei, seg] * aspl

            pltpu.sync_copy(hb, out_acc.at[trg_v.at[blk]], add=True)
            issue(blk + 2, b)

    for b in range(2):
        wait_three(b)
    plsc.subcore_barrier()
    pltpu.sync_copy(out_acc.at[pl.ds(sid * RPT, RPT), :],
                    out_hbm.at[cid, pl.ds(sid * RPT, RPT), :])


def _k4(src2, trg2, expe, rnorm, hp):
    mesh = plsc.VectorSubcoreMesh(core_axis_name="c", subcore_axis_name="s")
    f = pl.kernel(
        _k4_body,
        out_type=[
            jax.ShapeDtypeStruct((NC, N_PAD, KF), jnp.float32),
        ],
        mesh=mesh,
        compiler_params=pltpu.CompilerParams(use_tc_tiling_on_sc=False),
        scratch_types=[
            pltpu.VMEM((C0 + SLOP, EB4), jnp.int32),
            pltpu.VMEM((C0 + SLOP, EB4), jnp.int32),
            pltpu.VMEM((EB4, KP), jnp.float32),
            pltpu.VMEM((EB4, KP), jnp.float32),
            pltpu.VMEM((EB4, KP), jnp.float32),
            pltpu.VMEM((EB4, KP), jnp.float32),
            pltpu.VMEM((EB4, KF), jnp.float32),
            pltpu.VMEM((EB4, KF), jnp.float32),
            pltpu.VMEM_SHARED((N_PAD, KF), jnp.float32),
            pltpu.SemaphoreType.DMA,
            pltpu.SemaphoreType.DMA,
        ],
    )
    return f(src2, trg2, expe, rnorm, hp)


def _k5_body(p_ref, out_ref):
    out_ref[...] = p_ref[0] + p_ref[1]


def _k5(out_part):
    grid = N_PAD // 1024
    return pl.pallas_call(
        _k5_body,
        grid=(grid,),
        in_specs=[pl.BlockSpec((2, 1024, KF), lambda i: (0, i, 0))],
        out_specs=pl.BlockSpec((1024, KF), lambda i: (i, 0)),
        out_shape=jax.ShapeDtypeStruct((N_PAD, KF), jnp.float32),
    )(out_part)


def kernel(x, edge_index, W, b, a_src, a_trg):
    n = x.shape[0]
    e = edge_index.shape[1]
    e_tot = e + n
    self_loop = jnp.arange(n, dtype=edge_index.dtype)
    pad_idx = jnp.full((E_PAD - e_tot,), n, dtype=edge_index.dtype)
    src = jnp.concatenate([edge_index[0], self_loop, pad_idx])
    trg = jnp.concatenate([edge_index[1], self_loop, pad_idx])
    src2 = jnp.pad(src.reshape(NW, BPT, EB), ((0, 0), (0, SLOP), (0, 0)))
    trg2 = jnp.pad(trg.reshape(NW, BPT, EB), ((0, 0), (0, SLOP), (0, 0)))
    def _asym_pages(v):
        cut = NS * C0 * EB4
        p0 = v[:cut].reshape(NS, C0, EB4)
        p1 = v[cut:].reshape(NS, C1, EB4)
        p0 = jnp.pad(p0, ((0, 0), (0, SLOP), (0, 0)))
        p1 = jnp.pad(p1, ((0, 0), (0, C0 - C1 + SLOP), (0, 0)))
        return jnp.concatenate([p0, p1], axis=0)

    src4 = _asym_pages(src)
    trg4 = _asym_pages(trg)

    x_pad = jnp.pad(x, ((0, N_PAD - n), (0, 0)))
    Wt = W.T
    b2 = b.reshape(1, KF)
    rows = jnp.arange(KF)
    cols = rows // F_OUT
    A_src = jnp.zeros((KF, KP), jnp.float32).at[rows, cols].set(
        a_src.reshape(KF))
    A_trg = jnp.zeros((KF, KP), jnp.float32).at[rows, cols].set(
        a_trg.reshape(KF))

    hp, hs, ht, bound = _k1(x_pad, Wt, b2, A_src, A_trg)
    expe, norm_part = _k2(src2, trg2, hs, ht, bound)
    rnorm = _k3(norm_part)
    (out_part,) = _k4(src4, trg4, expe, rnorm, hp)
    out = _k5(out_part)
    return out[:n]
